# Initial kernel scaffold; baseline (speedup 1.0000x reference)
import jax
import jax.numpy as jnp
from jax.experimental import pallas as pl


def kernel(inputs, dur_w1, dur_b1, dur_g1, dur_be1, dur_w2, dur_b2, dur_g2, dur_be2, dur_lw, dur_lb, pit_w1, pit_b1, pit_g1, pit_be1, pit_w2, pit_b2, pit_g2, pit_be2, pit_lw, pit_lb, eng_w1, eng_b1, eng_g1, eng_be1, eng_w2, eng_b2, eng_g2, eng_be2, eng_lw, eng_lb):
    raise NotImplementedError("write your pallas kernel here")


# speedup vs baseline: 1.7452x; 1.7452x over previous
import jax
import jax.numpy as jnp
from jax.experimental import pallas as pl


def _adaptor_step(x_ref, w1_ref, b1_ref, g1_ref, be1_ref,
                  w2_ref, b2_ref, g2_ref, be2_ref, lw_ref, lb_ref,
                  out_ref, scal_ref):
    p = pl.program_id(1)
    x = x_ref[0]

    def conv3(h, w_ref):
        p0 = jnp.dot(h, w_ref[0, 0], preferred_element_type=jnp.float32)
        p1 = jnp.dot(h, w_ref[0, 1], preferred_element_type=jnp.float32)
        p2 = jnp.dot(h, w_ref[0, 2], preferred_element_type=jnp.float32)
        z = jnp.zeros((1, p0.shape[1]), jnp.float32)
        return (p1 + jnp.concatenate([z, p0[:-1]], axis=0)
                + jnp.concatenate([p2[1:], z], axis=0))

    def layer_norm(h, g, b):
        m = jnp.mean(h, axis=-1, keepdims=True)
        v = jnp.mean((h - m) ** 2, axis=-1, keepdims=True)
        return (h - m) * jax.lax.rsqrt(v + 1e-5) * g + b

    h = conv3(x, w1_ref) + b1_ref[0]
    h = jnp.maximum(h, 0.0)
    h = layer_norm(h, g1_ref[0], be1_ref[0])
    h = conv3(h, w2_ref) + b2_ref[0]
    h = jnp.maximum(h, 0.0)
    h = layer_norm(h, g2_ref[0], be2_ref[0])
    s = jnp.sum(h * lw_ref[0], axis=-1, keepdims=True) + lb_ref[0, 0]
    scal_ref[0, 0] = s

    @pl.when(p == 0)
    def _():
        out_ref[0] = x

    @pl.when(p != 0)
    def _():
        out_ref[0] = out_ref[0] + s


def kernel(inputs, dur_w1, dur_b1, dur_g1, dur_be1, dur_w2, dur_b2, dur_g2, dur_be2, dur_lw, dur_lb, pit_w1, pit_b1, pit_g1, pit_be1, pit_w2, pit_b2, pit_g2, pit_be2, pit_lw, pit_lb, eng_w1, eng_b1, eng_g1, eng_be1, eng_w2, eng_b2, eng_g2, eng_be2, eng_lw, eng_lb):
    B, L, C = inputs.shape
    F, _, K = dur_w1.shape

    w1 = jnp.stack([jnp.transpose(w, (2, 1, 0)) for w in (dur_w1, pit_w1, eng_w1)])
    w2 = jnp.stack([jnp.transpose(w, (2, 1, 0)) for w in (dur_w2, pit_w2, eng_w2)])
    b1 = jnp.stack([dur_b1, pit_b1, eng_b1])[:, None, :]
    g1 = jnp.stack([dur_g1, pit_g1, eng_g1])[:, None, :]
    be1 = jnp.stack([dur_be1, pit_be1, eng_be1])[:, None, :]
    b2 = jnp.stack([dur_b2, pit_b2, eng_b2])[:, None, :]
    g2 = jnp.stack([dur_g2, pit_g2, eng_g2])[:, None, :]
    be2 = jnp.stack([dur_be2, pit_be2, eng_be2])[:, None, :]
    lw = jnp.stack([dur_lw, pit_lw, eng_lw])
    lb = jnp.stack([dur_lb, pit_lb, eng_lb])[:, :, None]

    vec_spec = pl.BlockSpec((1, 1, F), lambda b, p: (p, 0, 0))
    outputs, scal = pl.pallas_call(
        _adaptor_step,
        grid=(B, 3),
        in_specs=[
            pl.BlockSpec((1, L, C), lambda b, p: (b, 0, 0)),
            pl.BlockSpec((1, K, C, F), lambda b, p: (p, 0, 0, 0)),
            vec_spec, vec_spec, vec_spec,
            pl.BlockSpec((1, K, F, F), lambda b, p: (p, 0, 0, 0)),
            vec_spec, vec_spec, vec_spec,
            vec_spec,
            pl.BlockSpec((1, 1, 1), lambda b, p: (p, 0, 0)),
        ],
        out_specs=[
            pl.BlockSpec((1, L, C), lambda b, p: (b, 0, 0)),
            pl.BlockSpec((1, 1, L, 1), lambda b, p: (p, b, 0, 0)),
        ],
        out_shape=[
            jax.ShapeDtypeStruct((B, L, C), jnp.float32),
            jax.ShapeDtypeStruct((3, B, L, 1), jnp.float32),
        ],
    )(inputs, w1, b1, g1, be1, w2, b2, g2, be2, lw, lb)

    return (outputs, scal[0], scal[1], scal[2])


# speedup vs baseline: 1.8247x; 1.0456x over previous
import jax
import jax.numpy as jnp
from jax.experimental import pallas as pl


def _adaptor_step(x_ref, w1_ref, b1_ref, g1_ref, be1_ref,
                  w2_ref, b2_ref, g2_ref, be2_ref, lw_ref, lb_ref,
                  out_ref, scal_ref):
    p = pl.program_id(1)
    x = x_ref[0]

    def conv3(h, w_ref):
        hb = h.astype(jnp.bfloat16)
        p0 = jnp.dot(hb, w_ref[0, 0], preferred_element_type=jnp.float32)
        p1 = jnp.dot(hb, w_ref[0, 1], preferred_element_type=jnp.float32)
        p2 = jnp.dot(hb, w_ref[0, 2], preferred_element_type=jnp.float32)
        z = jnp.zeros((1, p0.shape[1]), jnp.float32)
        return (p1 + jnp.concatenate([z, p0[:-1]], axis=0)
                + jnp.concatenate([p2[1:], z], axis=0))

    def layer_norm(h, g, b):
        m = jnp.mean(h, axis=-1, keepdims=True)
        v = jnp.mean((h - m) ** 2, axis=-1, keepdims=True)
        return (h - m) * jax.lax.rsqrt(v + 1e-5) * g + b

    h = conv3(x, w1_ref) + b1_ref[0]
    h = jnp.maximum(h, 0.0)
    h = layer_norm(h, g1_ref[0], be1_ref[0])
    h = conv3(h, w2_ref) + b2_ref[0]
    h = jnp.maximum(h, 0.0)
    h = layer_norm(h, g2_ref[0], be2_ref[0])
    s = jnp.sum(h * lw_ref[0], axis=-1, keepdims=True) + lb_ref[0, 0]
    scal_ref[0, 0] = s

    @pl.when(p == 0)
    def _():
        out_ref[0] = x

    @pl.when(p != 0)
    def _():
        out_ref[0] = out_ref[0] + s


def kernel(inputs, dur_w1, dur_b1, dur_g1, dur_be1, dur_w2, dur_b2, dur_g2, dur_be2, dur_lw, dur_lb, pit_w1, pit_b1, pit_g1, pit_be1, pit_w2, pit_b2, pit_g2, pit_be2, pit_lw, pit_lb, eng_w1, eng_b1, eng_g1, eng_be1, eng_w2, eng_b2, eng_g2, eng_be2, eng_lw, eng_lb):
    B, L, C = inputs.shape
    F, _, K = dur_w1.shape

    w1 = jnp.stack([jnp.transpose(w, (2, 1, 0)) for w in (dur_w1, pit_w1, eng_w1)]).astype(jnp.bfloat16)
    w2 = jnp.stack([jnp.transpose(w, (2, 1, 0)) for w in (dur_w2, pit_w2, eng_w2)]).astype(jnp.bfloat16)
    b1 = jnp.stack([dur_b1, pit_b1, eng_b1])[:, None, :]
    g1 = jnp.stack([dur_g1, pit_g1, eng_g1])[:, None, :]
    be1 = jnp.stack([dur_be1, pit_be1, eng_be1])[:, None, :]
    b2 = jnp.stack([dur_b2, pit_b2, eng_b2])[:, None, :]
    g2 = jnp.stack([dur_g2, pit_g2, eng_g2])[:, None, :]
    be2 = jnp.stack([dur_be2, pit_be2, eng_be2])[:, None, :]
    lw = jnp.stack([dur_lw, pit_lw, eng_lw])
    lb = jnp.stack([dur_lb, pit_lb, eng_lb])[:, :, None]

    vec_spec = pl.BlockSpec((1, 1, F), lambda b, p: (p, 0, 0))
    outputs, scal = pl.pallas_call(
        _adaptor_step,
        grid=(B, 3),
        in_specs=[
            pl.BlockSpec((1, L, C), lambda b, p: (b, 0, 0)),
            pl.BlockSpec((1, K, C, F), lambda b, p: (p, 0, 0, 0)),
            vec_spec, vec_spec, vec_spec,
            pl.BlockSpec((1, K, F, F), lambda b, p: (p, 0, 0, 0)),
            vec_spec, vec_spec, vec_spec,
            vec_spec,
            pl.BlockSpec((1, 1, 1), lambda b, p: (p, 0, 0)),
        ],
        out_specs=[
            pl.BlockSpec((1, L, C), lambda b, p: (b, 0, 0)),
            pl.BlockSpec((1, 1, L, 1), lambda b, p: (p, b, 0, 0)),
        ],
        out_shape=[
            jax.ShapeDtypeStruct((B, L, C), jnp.float32),
            jax.ShapeDtypeStruct((3, B, L, 1), jnp.float32),
        ],
    )(inputs, w1, b1, g1, be1, w2, b2, g2, be2, lw, lb)

    return (outputs, scal[0], scal[1], scal[2])
